# baseline (speedup 1.0000x reference)
import functools

import jax
import jax.numpy as jnp
from jax import lax
from jax.experimental import pallas as pl
from jax.experimental.pallas import tpu as pltpu
from jax.experimental.pallas import tpu_sc as plsc

ZCH = 1000000
EMB = 64
N = 16384 * 20
NS = 16
LANES = 16
NROW = N // 128
RPT = NROW // NS
HSL = 62504
HSL_LAST = ZCH - 15 * HSL
HPAD = 16 * HSL

_mesh = plsc.VectorSubcoreMesh(core_axis_name="c", subcore_axis_name="s")


@functools.partial(
    pl.kernel,
    out_type=[
        jax.ShapeDtypeStruct((NROW, 128), jnp.int32),
        jax.ShapeDtypeStruct((NROW, 128), jnp.int32),
        jax.ShapeDtypeStruct((ZCH,), jnp.float32),
        jax.ShapeDtypeStruct((ZCH,), jnp.float32),
    ],
    mesh=_mesh,
    compiler_params=pltpu.CompilerParams(use_tc_tiling_on_sc=False),
    scratch_types=[
        pltpu.VMEM((RPT, 128), jnp.int32),
        pltpu.VMEM((128,), jnp.float32),
        pltpu.VMEM_SHARED((HPAD,), jnp.float32),
        pltpu.SemaphoreType.DMA,
        pltpu.SemaphoreType.DMA,
        pltpu.SemaphoreType.DMA,
    ],
)
def _sc_hist(v0_hbm, v1_hbm, zeros_hbm, r0_hbm, r1_hbm, h0_hbm, h1_hbm,
             idx_v, ones_v, hist_s, zsem, wsem, ssem):
    cid = lax.axis_index("c")
    tid = lax.axis_index("s")
    hoff = tid * HSL

    for k in range(128 // LANES):
        ones_v[pl.ds(k * LANES, LANES)] = jnp.full((LANES,), 1.0, jnp.float32)

    pltpu.async_copy(zeros_hbm.at[pl.ds(0, HSL)],
                     hist_s.at[pl.ds(hoff, HSL)], zsem)

    def do_feature(v_hbm, r_hbm, h_hbm):
        base = tid * RPT
        pltpu.sync_copy(v_hbm.at[pl.ds(base, RPT)], idx_v)
        zmod = jnp.full((LANES,), ZCH, jnp.int32)

        def rem_row(i, _):
            for k in range(128 // LANES):
                sl = pl.ds(k * LANES, LANES)
                idx_v[i, sl] = lax.rem(idx_v[i, sl], zmod)
            return 0
        lax.fori_loop(0, RPT, rem_row, 0)
        w = pltpu.async_copy(idx_v, r_hbm.at[pl.ds(base, RPT)], wsem)

        pltpu.make_async_copy(zeros_hbm.at[pl.ds(0, HSL)],
                              hist_s.at[pl.ds(hoff, HSL)], zsem).wait()
        plsc.subcore_barrier()

        def blk(b, _):
            def fire(j, _):
                pltpu.async_copy(ones_v, hist_s.at[idx_v.at[b * 20 + j]],
                                 ssem, add=True)
                return 0
            lax.fori_loop(0, 20, fire, 0)

            def drain(j, _):
                pltpu.make_async_copy(ones_v, hist_s.at[idx_v.at[0]],
                                      ssem).wait()
                return 0
            lax.fori_loop(0, 20, drain, 0)
            return 0
        lax.fori_loop(0, RPT // 20, blk, 0)
        plsc.subcore_barrier()

        @pl.when(tid < NS - 1)
        def _():
            pltpu.sync_copy(hist_s.at[pl.ds(hoff, HSL)],
                            h_hbm.at[pl.ds(hoff, HSL)])
        @pl.when(tid == NS - 1)
        def _():
            pltpu.sync_copy(hist_s.at[pl.ds(hoff, HSL_LAST)],
                            h_hbm.at[pl.ds(hoff, HSL_LAST)])
        w.wait()

    @pl.when(cid == 0)
    def _():
        do_feature(v0_hbm, r0_hbm, h0_hbm)

    @pl.when(cid == 1)
    def _():
        do_feature(v1_hbm, r1_hbm, h1_hbm)


BLK = 49152
NBLK = (ZCH + BLK - 1) // BLK


def _rowsum_body(t0_ref, t1_ref, o0_ref, o1_ref):
    o0_ref[...] = jnp.sum(t0_ref[...], axis=0)
    o1_ref[...] = jnp.sum(t1_ref[...], axis=0)


_rowsum = pl.pallas_call(
    _rowsum_body,
    grid=(NBLK,),
    in_specs=[pl.BlockSpec((EMB, BLK), lambda i: (0, i)),
              pl.BlockSpec((EMB, BLK), lambda i: (0, i))],
    out_specs=[pl.BlockSpec((BLK,), lambda i: (i,)),
               pl.BlockSpec((BLK,), lambda i: (i,))],
    out_shape=[jax.ShapeDtypeStruct((ZCH,), jnp.float32)] * 2,
)

DBLK = 131072
DNBLK = (ZCH + DBLK - 1) // DBLK


def _dot_body(h0_ref, r0_ref, h1_ref, r1_ref, acc_ref):
    i = pl.program_id(0)

    @pl.when(i == 0)
    def _():
        acc_ref[...] = jnp.zeros_like(acc_ref)

    prod = h0_ref[...] * r0_ref[...] + h1_ref[...] * r1_ref[...]
    p2 = prod.reshape(DBLK // 128, 128)

    def tree_sum(x):
        s = x[0:8]
        for k in range(1, DBLK // 1024):
            s = s + x[k * 8:(k + 1) * 8]
        return s

    @pl.when(i < DNBLK - 1)
    def _():
        acc_ref[...] += tree_sum(p2)

    @pl.when(i == DNBLK - 1)
    def _():
        flat = (lax.broadcasted_iota(jnp.int32, (DBLK // 128, 128), 0) * 128
                + lax.broadcasted_iota(jnp.int32, (DBLK // 128, 128), 1))
        acc_ref[...] += tree_sum(jnp.where(flat < ZCH - i * DBLK, p2, 0.0))


_dot = pl.pallas_call(
    _dot_body,
    grid=(DNBLK,),
    in_specs=[pl.BlockSpec((DBLK,), lambda i: (i,))] * 4,
    out_specs=pl.BlockSpec((8, 128), lambda i: (0, 0)),
    out_shape=jax.ShapeDtypeStruct((8, 128), jnp.float32),
)


def kernel(values_0, values_1, table_0, table_1):
    v0 = values_0.reshape(NROW, 128)
    v1 = values_1.reshape(NROW, 128)
    zeros = jnp.zeros((HSL,), jnp.float32)
    r0, r1, h0, h1 = _sc_hist(v0, v1, zeros)
    rs0, rs1 = _rowsum(table_0.T, table_1.T)
    acc = _dot(h0, rs0, h1, rs1)
    loss = jnp.sum(acc) / jnp.float32(2 * N * EMB)
    return (loss, r0.reshape(N), r1.reshape(N))


# speedup vs baseline: 1.0112x; 1.0112x over previous
import functools

import jax
import jax.numpy as jnp
from jax import lax
from jax.experimental import pallas as pl
from jax.experimental.pallas import tpu as pltpu
from jax.experimental.pallas import tpu_sc as plsc

ZCH = 1000000
EMB = 64
N = 16384 * 20
NS = 16
LANES = 16
NROW = N // 128
RPT = NROW // NS
HSL = 62504
HSL_LAST = ZCH - 15 * HSL
HPAD = 16 * HSL

_mesh = plsc.VectorSubcoreMesh(core_axis_name="c", subcore_axis_name="s")


@functools.partial(
    pl.kernel,
    out_type=[
        jax.ShapeDtypeStruct((NROW, 128), jnp.int32),
        jax.ShapeDtypeStruct((NROW, 128), jnp.int32),
        jax.ShapeDtypeStruct((ZCH,), jnp.float32),
        jax.ShapeDtypeStruct((ZCH,), jnp.float32),
    ],
    mesh=_mesh,
    compiler_params=pltpu.CompilerParams(use_tc_tiling_on_sc=False),
    scratch_types=[
        pltpu.VMEM((RPT, 128), jnp.int32),
        pltpu.VMEM((128,), jnp.float32),
        pltpu.VMEM_SHARED((HPAD,), jnp.float32),
        pltpu.SemaphoreType.DMA,
        pltpu.SemaphoreType.DMA,
        pltpu.SemaphoreType.DMA,
    ],
)
def _sc_hist(v0_hbm, v1_hbm, zeros_hbm, r0_hbm, r1_hbm, h0_hbm, h1_hbm,
             idx_v, ones_v, hist_s, zsem, wsem, ssem):
    cid = lax.axis_index("c")
    tid = lax.axis_index("s")
    hoff = tid * HSL

    for k in range(128 // LANES):
        ones_v[pl.ds(k * LANES, LANES)] = jnp.full((LANES,), 1.0, jnp.float32)

    pltpu.async_copy(zeros_hbm.at[pl.ds(0, HSL)],
                     hist_s.at[pl.ds(hoff, HSL)], zsem)

    def do_feature(v_hbm, r_hbm, h_hbm):
        base = tid * RPT
        pltpu.sync_copy(v_hbm.at[pl.ds(base, RPT)], idx_v)
        zmod = jnp.full((LANES,), ZCH, jnp.int32)

        def rem_row(i, _):
            for k in range(128 // LANES):
                sl = pl.ds(k * LANES, LANES)
                idx_v[i, sl] = lax.rem(idx_v[i, sl], zmod)
            return 0
        lax.fori_loop(0, RPT, rem_row, 0)
        w = pltpu.async_copy(idx_v, r_hbm.at[pl.ds(base, RPT)], wsem)

        pltpu.make_async_copy(zeros_hbm.at[pl.ds(0, HSL)],
                              hist_s.at[pl.ds(hoff, HSL)], zsem).wait()
        plsc.subcore_barrier()

        def blk(b, _):
            def fire(j, _):
                pltpu.async_copy(ones_v, hist_s.at[idx_v.at[b * 20 + j]],
                                 ssem, add=True)
                return 0
            lax.fori_loop(0, 20, fire, 0)

            def drain(j, _):
                pltpu.make_async_copy(ones_v, hist_s.at[idx_v.at[0]],
                                      ssem).wait()
                return 0
            lax.fori_loop(0, 20, drain, 0)
            return 0
        lax.fori_loop(0, RPT // 20, blk, 0)
        plsc.subcore_barrier()

        @pl.when(tid < NS - 1)
        def _():
            pltpu.sync_copy(hist_s.at[pl.ds(hoff, HSL)],
                            h_hbm.at[pl.ds(hoff, HSL)])
        @pl.when(tid == NS - 1)
        def _():
            pltpu.sync_copy(hist_s.at[pl.ds(hoff, HSL_LAST)],
                            h_hbm.at[pl.ds(hoff, HSL_LAST)])
        w.wait()

    @pl.when(cid == 0)
    def _():
        do_feature(v0_hbm, r0_hbm, h0_hbm)

    @pl.when(cid == 1)
    def _():
        do_feature(v1_hbm, r1_hbm, h1_hbm)


BLK = 20480
NBLK = (ZCH + BLK - 1) // BLK


def _rowsum_body(t0_ref, t1_ref, o0_ref, o1_ref):
    o0_ref[...] = jnp.sum(t0_ref[...], axis=0)
    o1_ref[...] = jnp.sum(t1_ref[...], axis=0)


_rowsum = pl.pallas_call(
    _rowsum_body,
    grid=(NBLK,),
    in_specs=[pl.BlockSpec((EMB, BLK), lambda i: (0, i)),
              pl.BlockSpec((EMB, BLK), lambda i: (0, i))],
    out_specs=[pl.BlockSpec((BLK,), lambda i: (i,)),
               pl.BlockSpec((BLK,), lambda i: (i,))],
    out_shape=[jax.ShapeDtypeStruct((ZCH,), jnp.float32)] * 2,
)

DBLK = 131072
DNBLK = (ZCH + DBLK - 1) // DBLK


def _dot_body(h0_ref, r0_ref, h1_ref, r1_ref, out_ref, acc_ref):
    i = pl.program_id(0)

    @pl.when(i == 0)
    def _():
        acc_ref[...] = jnp.zeros_like(acc_ref)

    prod = h0_ref[...] * r0_ref[...] + h1_ref[...] * r1_ref[...]
    p2 = prod.reshape(DBLK // 128, 128)

    def tree_sum(x):
        s = x[0:8]
        for k in range(1, DBLK // 1024):
            s = s + x[k * 8:(k + 1) * 8]
        return s

    @pl.when(i < DNBLK - 1)
    def _():
        acc_ref[...] += tree_sum(p2)

    @pl.when(i == DNBLK - 1)
    def _():
        flat = (lax.broadcasted_iota(jnp.int32, (DBLK // 128, 128), 0) * 128
                + lax.broadcasted_iota(jnp.int32, (DBLK // 128, 128), 1))
        acc_ref[...] += tree_sum(jnp.where(flat < ZCH - i * DBLK, p2, 0.0))
        out_ref[0, 0] = jnp.sum(acc_ref[...])


_dot = pl.pallas_call(
    _dot_body,
    grid=(DNBLK,),
    in_specs=[pl.BlockSpec((DBLK,), lambda i: (i,))] * 4,
    out_specs=pl.BlockSpec(memory_space=pltpu.SMEM),
    out_shape=jax.ShapeDtypeStruct((1, 1), jnp.float32),
    scratch_shapes=[pltpu.VMEM((8, 128), jnp.float32)],
)


def kernel(values_0, values_1, table_0, table_1):
    v0 = values_0.reshape(NROW, 128)
    v1 = values_1.reshape(NROW, 128)
    zeros = jnp.zeros((HSL,), jnp.float32)
    r0, r1, h0, h1 = _sc_hist(v0, v1, zeros)
    rs0, rs1 = _rowsum(table_0.T, table_1.T)
    acc = _dot(h0, rs0, h1, rs1)
    loss = acc[0, 0] / jnp.float32(2 * N * EMB)
    return (loss, r0.reshape(N), r1.reshape(N))


# speedup vs baseline: 1.0120x; 1.0007x over previous
import functools

import jax
import jax.numpy as jnp
from jax import lax
from jax.experimental import pallas as pl
from jax.experimental.pallas import tpu as pltpu
from jax.experimental.pallas import tpu_sc as plsc

ZCH = 1000000
EMB = 64
N = 16384 * 20
NS = 16
LANES = 16
NROW = N // 128
RPT = NROW // NS
HSL = 62504
HSL_LAST = ZCH - 15 * HSL
HPAD = 16 * HSL

_mesh = plsc.VectorSubcoreMesh(core_axis_name="c", subcore_axis_name="s")


@functools.partial(
    pl.kernel,
    out_type=[
        jax.ShapeDtypeStruct((NROW, 128), jnp.int32),
        jax.ShapeDtypeStruct((NROW, 128), jnp.int32),
        jax.ShapeDtypeStruct((ZCH,), jnp.float32),
        jax.ShapeDtypeStruct((ZCH,), jnp.float32),
    ],
    mesh=_mesh,
    compiler_params=pltpu.CompilerParams(use_tc_tiling_on_sc=False),
    scratch_types=[
        pltpu.VMEM((RPT, 128), jnp.int32),
        pltpu.VMEM((128,), jnp.float32),
        pltpu.VMEM_SHARED((HPAD,), jnp.float32),
        pltpu.SemaphoreType.DMA,
        pltpu.SemaphoreType.DMA,
        pltpu.SemaphoreType.DMA,
    ],
)
def _sc_hist(v0_hbm, v1_hbm, zeros_hbm, r0_hbm, r1_hbm, h0_hbm, h1_hbm,
             idx_v, ones_v, hist_s, zsem, wsem, ssem):
    cid = lax.axis_index("c")
    tid = lax.axis_index("s")
    hoff = tid * HSL

    for k in range(128 // LANES):
        ones_v[pl.ds(k * LANES, LANES)] = jnp.full((LANES,), 1.0, jnp.float32)

    pltpu.async_copy(zeros_hbm.at[pl.ds(0, HSL)],
                     hist_s.at[pl.ds(hoff, HSL)], zsem)

    base = tid * RPT

    @pl.when(cid == 0)
    def _():
        pltpu.sync_copy(v0_hbm.at[pl.ds(base, RPT)], idx_v)

    @pl.when(cid == 1)
    def _():
        pltpu.sync_copy(v1_hbm.at[pl.ds(base, RPT)], idx_v)

    zmod = jnp.full((LANES,), ZCH, jnp.int32)

    def rem_row(i, _):
        for k in range(128 // LANES):
            sl = pl.ds(k * LANES, LANES)
            idx_v[i, sl] = lax.rem(idx_v[i, sl], zmod)
        return 0
    lax.fori_loop(0, RPT, rem_row, 0)

    @pl.when(cid == 0)
    def _():
        pltpu.async_copy(idx_v, r0_hbm.at[pl.ds(base, RPT)], wsem)

    @pl.when(cid == 1)
    def _():
        pltpu.async_copy(idx_v, r1_hbm.at[pl.ds(base, RPT)], wsem)

    pltpu.make_async_copy(zeros_hbm.at[pl.ds(0, HSL)],
                          hist_s.at[pl.ds(hoff, HSL)], zsem).wait()
    plsc.subcore_barrier()

    def blk(b, _):
        def fire(j, _):
            pltpu.async_copy(ones_v, hist_s.at[idx_v.at[b * 20 + j]],
                             ssem, add=True)
            return 0
        lax.fori_loop(0, 20, fire, 0)

        def drain(j, _):
            pltpu.make_async_copy(ones_v, hist_s.at[idx_v.at[0]],
                                  ssem).wait()
            return 0
        lax.fori_loop(0, 20, drain, 0)
        return 0
    lax.fori_loop(0, RPT // 20, blk, 0)
    plsc.subcore_barrier()

    hlen_last = HSL_LAST

    @pl.when((cid == 0) & (tid < NS - 1))
    def _():
        pltpu.sync_copy(hist_s.at[pl.ds(hoff, HSL)],
                        h0_hbm.at[pl.ds(hoff, HSL)])

    @pl.when((cid == 0) & (tid == NS - 1))
    def _():
        pltpu.sync_copy(hist_s.at[pl.ds(hoff, hlen_last)],
                        h0_hbm.at[pl.ds(hoff, hlen_last)])

    @pl.when((cid == 1) & (tid < NS - 1))
    def _():
        pltpu.sync_copy(hist_s.at[pl.ds(hoff, HSL)],
                        h1_hbm.at[pl.ds(hoff, HSL)])

    @pl.when((cid == 1) & (tid == NS - 1))
    def _():
        pltpu.sync_copy(hist_s.at[pl.ds(hoff, hlen_last)],
                        h1_hbm.at[pl.ds(hoff, hlen_last)])

    pltpu.make_async_copy(idx_v, r0_hbm.at[pl.ds(base, RPT)], wsem).wait()


BLK = 20480
NBLK = (ZCH + BLK - 1) // BLK


def _rowsum_body(t0_ref, t1_ref, o0_ref, o1_ref):
    o0_ref[...] = jnp.sum(t0_ref[...], axis=0)
    o1_ref[...] = jnp.sum(t1_ref[...], axis=0)


_rowsum = pl.pallas_call(
    _rowsum_body,
    grid=(NBLK,),
    in_specs=[pl.BlockSpec((EMB, BLK), lambda i: (0, i)),
              pl.BlockSpec((EMB, BLK), lambda i: (0, i))],
    out_specs=[pl.BlockSpec((BLK,), lambda i: (i,)),
               pl.BlockSpec((BLK,), lambda i: (i,))],
    out_shape=[jax.ShapeDtypeStruct((ZCH,), jnp.float32)] * 2,
)

DBLK = 131072
DNBLK = (ZCH + DBLK - 1) // DBLK


def _dot_body(h0_ref, r0_ref, h1_ref, r1_ref, out_ref, acc_ref):
    i = pl.program_id(0)

    @pl.when(i == 0)
    def _():
        acc_ref[...] = jnp.zeros_like(acc_ref)

    prod = h0_ref[...] * r0_ref[...] + h1_ref[...] * r1_ref[...]
    p2 = prod.reshape(DBLK // 128, 128)

    def tree_sum(x):
        s = x[0:8]
        for k in range(1, DBLK // 1024):
            s = s + x[k * 8:(k + 1) * 8]
        return s

    @pl.when(i < DNBLK - 1)
    def _():
        acc_ref[...] += tree_sum(p2)

    @pl.when(i == DNBLK - 1)
    def _():
        flat = (lax.broadcasted_iota(jnp.int32, (DBLK // 128, 128), 0) * 128
                + lax.broadcasted_iota(jnp.int32, (DBLK // 128, 128), 1))
        acc_ref[...] += tree_sum(jnp.where(flat < ZCH - i * DBLK, p2, 0.0))
        out_ref[0, 0] = jnp.sum(acc_ref[...])


_dot = pl.pallas_call(
    _dot_body,
    grid=(DNBLK,),
    in_specs=[pl.BlockSpec((DBLK,), lambda i: (i,))] * 4,
    out_specs=pl.BlockSpec(memory_space=pltpu.SMEM),
    out_shape=jax.ShapeDtypeStruct((1, 1), jnp.float32),
    scratch_shapes=[pltpu.VMEM((8, 128), jnp.float32)],
)


def kernel(values_0, values_1, table_0, table_1):
    v0 = values_0.reshape(NROW, 128)
    v1 = values_1.reshape(NROW, 128)
    zeros = jnp.zeros((HSL,), jnp.float32)
    r0, r1, h0, h1 = _sc_hist(v0, v1, zeros)
    rs0, rs1 = _rowsum(table_0.T, table_1.T)
    acc = _dot(h0, rs0, h1, rs1)
    loss = acc[0, 0] / jnp.float32(2 * N * EMB)
    return (loss, r0.reshape(N), r1.reshape(N))
